# baseline (speedup 1.0000x reference)
import jax
import jax.numpy as jnp
from jax.experimental import pallas as pl
from jax.experimental.pallas import tpu as pltpu

_BIN_NUM = 100
_DIM = 128
_BIN_ALPHA = 1.0
_MASK_TOKEN_ID = -10.0
_PAD_TOKEN_ID = -20.0

_BLOCK_R = 2048


def _fused_kernel(x_ref, w1_ref, b1_ref, w2_ref, b2_ref, emb_ref,
                  emb_mask_ref, emb_pad_ref, out_ref):
    x = x_ref[...]
    h = x * w1_ref[...] + b1_ref[...]
    h = jnp.where(h >= 0.0, h, 0.1 * h)
    h_cross = jnp.dot(h, w2_ref[...], preferred_element_type=jnp.float32)
    h = _BIN_ALPHA * h + h_cross + b2_ref[...]
    h = h - jnp.max(h, axis=-1, keepdims=True)
    e = jnp.exp(h)
    w = e / jnp.sum(e, axis=-1, keepdims=True)
    out = jnp.dot(w, emb_ref[...], preferred_element_type=jnp.float32)
    out = jnp.where(x == _MASK_TOKEN_ID, emb_mask_ref[...], out)
    out = jnp.where(x == _PAD_TOKEN_ID, emb_pad_ref[...], out)
    out_ref[...] = out


def kernel(x, W1, b1, W2, b2, emb, emb_mask, emb_pad):
    B, L, _ = x.shape
    rows = B * L
    x2 = x.reshape(rows, 1)
    grid = rows // _BLOCK_R

    const_spec = lambda shape: pl.BlockSpec(shape, lambda i: (0, 0))
    out2 = pl.pallas_call(
        _fused_kernel,
        grid=(grid,),
        in_specs=[
            pl.BlockSpec((_BLOCK_R, 1), lambda i: (i, 0)),
            const_spec((1, _BIN_NUM)),
            const_spec((1, _BIN_NUM)),
            const_spec((_BIN_NUM, _BIN_NUM)),
            const_spec((1, _BIN_NUM)),
            const_spec((_BIN_NUM, _DIM)),
            const_spec((1, _DIM)),
            const_spec((1, _DIM)),
        ],
        out_specs=pl.BlockSpec((_BLOCK_R, _DIM), lambda i: (i, 0)),
        out_shape=jax.ShapeDtypeStruct((rows, _DIM), jnp.float32),
        compiler_params=pltpu.CompilerParams(
            dimension_semantics=("arbitrary",),
        ),
    )(x2, W1, b1.reshape(1, _BIN_NUM), W2, b2.reshape(1, _BIN_NUM),
      emb, emb_mask, emb_pad)
    return out2.reshape(B, L, _DIM)
